# baseline (speedup 1.0000x reference)
import functools

import jax
import jax.numpy as jnp
from jax import lax
from jax.experimental import pallas as pl
from jax.experimental.pallas import tpu as pltpu
from jax.experimental.pallas import tpu_sc as plsc

N = 10000
NP = 10112
SLICE = NP // 16
MAXN = 9993
E = 160000
CHUNK = 128
EP = 163840
NROWS = EP // CHUNK
RPT = NROWS // 32
SUP = 8
DUMP = N

R8 = NP * 8 // 128
R16 = NP * 16 // 128
R32 = NP * 32 // 128


@functools.lru_cache(maxsize=None)
def _mesh():
    return plsc.VectorSubcoreMesh(core_axis_name="c", subcore_axis_name="s")


def _relu6(v):
    return jnp.clip(v, 0.0, 6.0)


def _mm(a, b):
    return jax.lax.dot_general(a, b, (((1,), (0,)), ((), ())),
                               preferred_element_type=jnp.float32)


@functools.lru_cache(maxsize=None)
def _make_deg():
    @functools.partial(
        pl.kernel,
        out_type=jax.ShapeDtypeStruct((2, NP, 16), jnp.float32),
        mesh=_mesh(),
        scratch_types=[
            pltpu.VMEM((RPT, CHUNK), jnp.int32),
            pltpu.VMEM((CHUNK, 16), jnp.float32),
            pltpu.VMEM_SHARED((NP, 16), jnp.float32),
            pltpu.SemaphoreType.DMA,
        ],
        compiler_params=pltpu.CompilerParams(use_tc_tiling_on_sc=False),
    )
    def deg_kernel(dstr_hbm, ones_hbm, zeros_hbm, out_hbm, didx, ones_v, acc,
                   sem):
        c = lax.axis_index("c")
        s = lax.axis_index("s")
        tile = s * 2 + c
        pltpu.sync_copy(zeros_hbm.at[pl.ds(s * SLICE, SLICE)],
                        acc.at[pl.ds(s * SLICE, SLICE)])
        pltpu.sync_copy(dstr_hbm.at[pl.ds(tile * RPT, RPT)], didx)
        pltpu.sync_copy(ones_hbm, ones_v)
        plsc.subcore_barrier()
        handles = []
        for g in range(RPT):
            handles.append(
                pltpu.async_copy(ones_v, acc.at[didx.at[g]], sem, add=True))
            if len(handles) > 8:
                handles.pop(0).wait()
        for h in handles:
            h.wait()
        plsc.subcore_barrier()
        pltpu.sync_copy(acc.at[pl.ds(s * SLICE, SLICE)],
                        out_hbm.at[c, pl.ds(s * SLICE, SLICE)])

    return deg_kernel


@functools.lru_cache(maxsize=None)
def _make_segsum(F):
    @functools.partial(
        pl.kernel,
        out_type=jax.ShapeDtypeStruct((2, NP, F), jnp.float32),
        mesh=_mesh(),
        scratch_types=[
            pltpu.VMEM((RPT * CHUNK,), jnp.int32),
            pltpu.VMEM((RPT, CHUNK), jnp.int32),
            pltpu.VMEM((SUP * CHUNK, F), jnp.float32),
            pltpu.VMEM((SUP * CHUNK, F), jnp.float32),
            pltpu.VMEM((SUP * CHUNK, F), jnp.float32),
            pltpu.VMEM_SHARED((NP, F), jnp.float32),
            pltpu.SemaphoreType.DMA,
            pltpu.SemaphoreType.DMA,
            pltpu.SemaphoreType.DMA,
            pltpu.SemaphoreType.DMA,
            pltpu.SemaphoreType.DMA,
            pltpu.SemaphoreType.DMA,
        ],
        compiler_params=pltpu.CompilerParams(use_tc_tiling_on_sc=False),
    )
    def seg_kernel(y_hbm, srcf_hbm, spad_hbm, dstr_hbm, zeros_hbm, out_hbm,
                   sidx, didx, buf_a, buf_b, buf_c, acc,
                   gs_a, gs_b, gs_c, ss_a, ss_b, ss_c):
        c = lax.axis_index("c")
        s = lax.axis_index("s")
        tile = s * 2 + c

        pltpu.sync_copy(zeros_hbm.at[pl.ds(s * SLICE, SLICE)],
                        acc.at[pl.ds(s * SLICE, SLICE)])
        pltpu.sync_copy(srcf_hbm.at[pl.ds(tile * (E // 32), E // 32)],
                        sidx.at[pl.ds(0, E // 32)])
        pltpu.sync_copy(spad_hbm, sidx.at[pl.ds(E // 32, EP // 32 - E // 32)])
        pltpu.sync_copy(dstr_hbm.at[pl.ds(tile * RPT, RPT)], didx)
        plsc.subcore_barrier()

        banks = ((buf_a, gs_a, ss_a), (buf_b, gs_b, ss_b),
                 (buf_c, gs_c, ss_c))
        NB = RPT // SUP

        def fire_gather(bank, base):
            buf, gs, _ = banks[bank]
            return [pltpu.async_copy(
                y_hbm.at[sidx.at[pl.ds(base * CHUNK, SUP * CHUNK)]],
                buf, gs)]

        scat_pend = {0: [], 1: [], 2: []}
        hg = {0: fire_gather(0, 0), 1: fire_gather(1, SUP), 2: None}
        for sc in range(NB):
            bank = sc % 3
            buf, _, ss = banks[bank]
            base = sc * SUP
            for h in hg[bank]:
                h.wait()
            if sc + 2 < NB:
                nbank = (sc + 2) % 3
                for h in scat_pend[nbank]:
                    h.wait()
                scat_pend[nbank] = []
                hg[nbank] = fire_gather(nbank, base + 2 * SUP)
            scat_pend[bank] = [
                pltpu.async_copy(buf.at[pl.ds(j * CHUNK, CHUNK)],
                                 acc.at[didx.at[base + j]],
                                 ss, add=True) for j in range(SUP)]
        for b in (0, 1, 2):
            for h in scat_pend[b]:
                h.wait()
        plsc.subcore_barrier()
        pltpu.sync_copy(acc.at[pl.ds(s * SLICE, SLICE)],
                        out_hbm.at[c, pl.ds(s * SLICE, SLICE)])

    return seg_kernel


def _deg_sc(*a):
    return _make_deg()(*a)


def _seg16(*a):
    return _make_segsum(16)(*a)


def _seg32(*a):
    return _make_segsum(32)(*a)


def _tc1_body(feat_ref, degs_ref, w0k_ref, b0k_ref, w1k_ref, t32_ref,
              xw1_ref, y1_ref, dinv16_ref, dinv32_ref):
    deg = degs_ref[0] + degs_ref[1] + 1.0
    dinv16 = lax.rsqrt(deg)
    x0 = _relu6(_mm(feat_ref[...], w0k_ref[...]) + b0k_ref[...])
    xw1 = jnp.reshape(_mm(x0, w1k_ref[...]), (R16, 128))
    deg32 = jnp.reshape(_mm(deg, t32_ref[...]), (R32, 128))
    dinv32 = lax.rsqrt(deg32)
    xw1_ref[...] = xw1
    y1_ref[...] = xw1 * dinv16
    dinv16_ref[...] = dinv16
    dinv32_ref[...] = dinv32


def _tc1(featp, degsp, W0k, b0k, Wg1k, T32k):
    return pl.pallas_call(
        _tc1_body,
        out_shape=[
            jax.ShapeDtypeStruct((R16, 128), jnp.float32),
            jax.ShapeDtypeStruct((R16, 128), jnp.float32),
            jax.ShapeDtypeStruct((R16, 128), jnp.float32),
            jax.ShapeDtypeStruct((R32, 128), jnp.float32),
        ],
    )(featp, degsp, W0k, b0k, Wg1k, T32k)


def _tc_layer_body(z_ref, xw_ref, dinv_ref, dinvn_ref, bk_ref, wnk_ref,
                   xwn_ref, yn_ref):
    dinv = dinv_ref[...]
    x = _relu6((z_ref[0] + z_ref[1] + xw_ref[...] * dinv) * dinv
               + bk_ref[...])
    xwn = jnp.reshape(_mm(x, wnk_ref[...]), (R32, 128))
    xwn_ref[...] = xwn
    yn_ref[...] = xwn * dinvn_ref[...]


def _tc_layer(zp, xwp, dinvp, dinvnp, bk, Wnk):
    return pl.pallas_call(
        _tc_layer_body,
        out_shape=[
            jax.ShapeDtypeStruct((R32, 128), jnp.float32),
            jax.ShapeDtypeStruct((R32, 128), jnp.float32),
        ],
    )(zp, xwp, dinvp, dinvnp, bk, Wnk)


def _head_body(z_ref, xw_ref, dinv_ref, bgk_ref, mask_ref,
               ws1k_ref, bs1k_ref, ws2k_ref, bs2k_ref,
               wt1ak_ref, bt1k_ref, wt1b4_ref, wt2k_ref, bt2k_ref,
               sp_ref, bsrc_ref, tp_ref, btgt_ref):
    dinv = dinv_ref[...]
    x = _relu6((z_ref[0] + z_ref[1] + xw_ref[...] * dinv) * dinv
               + bgk_ref[...])
    node = (lax.broadcasted_iota(jnp.int32, (R32, 4), 0) * 4
            + lax.broadcasted_iota(jnp.int32, (R32, 4), 1))
    valid = node < N

    hh = _relu6(_mm(x, ws1k_ref[...]) + bs1k_ref[...])
    sl = _mm(hh, ws2k_ref[...]) + bs2k_ref[...]
    slv = jnp.where(valid, sl, -1e30)
    e = jnp.where(valid, jnp.exp(sl - jnp.max(slv)), 0.0)
    sp = e / jnp.sum(e)
    m = mask_ref[...] > 0.0
    sp_ref[...] = jnp.where(m, 0.0, sp)
    am = jnp.where(valid, jnp.where(m, -1.0, sp), -2.0)
    best = jnp.min(jnp.where(am == jnp.max(am), node, NP))
    bsrc_ref[...] = jnp.full((1, 1), best, jnp.int32)

    rowsel = lax.broadcasted_iota(jnp.int32, (R32, 128), 0) == (best // 4)
    xsrow = jnp.sum(jnp.where(rowsel, x, 0.0), axis=0, keepdims=True)
    gsel = (lax.broadcasted_iota(jnp.int32, (1, 128), 1) // 32) == (best % 4)
    xsw = _mm(jnp.where(gsel, xsrow, 0.0), wt1b4_ref[...])
    xsw4 = jnp.concatenate([xsw, xsw, xsw, xsw], axis=1)
    th = _relu6(_mm(x, wt1ak_ref[...]) + xsw4 + bt1k_ref[...])
    tl = _mm(th, wt2k_ref[...]) + bt2k_ref[...]
    tlv = jnp.where(valid, tl, -1e30)
    te = jnp.where(valid, jnp.exp(tl - jnp.max(tlv)), 0.0)
    tp = te / jnp.sum(te)
    tvalid = node < MAXN
    tp_ref[...] = jnp.where(tvalid, tp, 0.0)
    am2 = jnp.where(tvalid, tp, -1.0)
    best2 = jnp.min(jnp.where(am2 == jnp.max(am2), node, NP))
    btgt_ref[...] = jnp.full((1, 1), best2, jnp.int32)


def _head(zp, xwp, dinvp, bgk, maskp, Ws1k, bs1k, Ws2k, bs2k,
          Wt1ak, bt1k, Wt1b4, Wt2k, bt2k):
    return pl.pallas_call(
        _head_body,
        out_shape=[
            jax.ShapeDtypeStruct((R32, 4), jnp.float32),
            jax.ShapeDtypeStruct((1, 1), jnp.int32),
            jax.ShapeDtypeStruct((R32, 4), jnp.float32),
            jax.ShapeDtypeStruct((1, 1), jnp.int32),
        ],
    )(zp, xwp, dinvp, bgk, maskp, Ws1k, bs1k, Ws2k, bs2k,
      Wt1ak, bt1k, Wt1b4, Wt2k, bt2k)


def _kron(k, W):
    return jnp.kron(jnp.eye(k, dtype=jnp.float32), W)


def kernel(feat, edge_index, mask_candidate_set,
           W0, b0, Wg1, bg1, Wg2, bg2, Wg3, bg3,
           Ws1, bs1, Ws2, bs2, Wt1, bt1, Wt2, bt2):
    src = edge_index[0].astype(jnp.int32)
    dst = edge_index[1].astype(jnp.int32)
    spad = DUMP + (jnp.arange(EP // 32 - E // 32, dtype=jnp.int32) % (NP - N))
    dump = jnp.broadcast_to(spad, (32, EP // 32 - E // 32))
    dstr = jnp.concatenate([dst.reshape(32, E // 32), dump],
                           axis=1).reshape(NROWS, CHUNK)

    featp = jnp.pad(feat, ((0, NP - N), (0, 1))).reshape(R8, 128)
    maskp = jnp.pad(mask_candidate_set.astype(jnp.float32),
                    (0, NP - N)).reshape(R32, 4)
    ones_c = jnp.ones((CHUNK, 16), jnp.float32)
    z16 = jnp.zeros((NP, 16), jnp.float32)
    z32 = jnp.zeros((NP, 32), jnp.float32)

    W0p = jnp.pad(W0, ((0, 1), (0, 0)))
    W0k = _kron(16, W0p)
    b0k = jnp.tile(b0, 16)
    Wg1k = _kron(16, Wg1)
    bg1k = jnp.tile(bg1, 8)
    T = jnp.zeros((16, 32), jnp.float32).at[0, :].set(1.0)
    T32k = _kron(8, T)
    Wg2p = jnp.pad(Wg2, ((0, 0), (0, 8)))
    Wg2k = _kron(8, Wg2p)
    bg2k = jnp.tile(jnp.pad(bg2, (0, 8)), 4)
    Wg3p = jnp.pad(Wg3, ((0, 8), (0, 0)))
    Wg3k = _kron(4, Wg3p)
    bg3k = jnp.tile(bg3, 4)
    Ws1k = _kron(4, Ws1)
    bs1k = jnp.tile(bs1, 4)
    Ws2k = _kron(4, Ws2)
    bs2k = jnp.tile(bs2, 4)
    Wt1ak = _kron(4, Wt1[:32])
    bt1k = jnp.tile(bt1, 4)
    Wt1b4 = jnp.concatenate([Wt1[32:]] * 4, axis=0)
    Wt2k = _kron(4, Wt2)
    bt2k = jnp.tile(bt2, 4)

    degs = _deg_sc(dstr, ones_c, z16)
    degsp = degs.reshape(2, R16, 128)
    xw1, y1, dinv16, dinv32 = _tc1(featp, degsp, W0k, b0k, Wg1k, T32k)
    z1 = _seg16(y1.reshape(NP, 16), src, spad, dstr, z16).reshape(2, R16, 128)
    xw2, y2 = _tc_layer(z1, xw1, dinv16, dinv32, bg1k, Wg2k)
    z2 = _seg32(y2.reshape(NP, 32), src, spad, dstr, z32).reshape(2, R32, 128)
    xw3, y3 = _tc_layer(z2, xw2, dinv32, dinv32, bg2k, Wg3k)
    z3 = _seg32(y3.reshape(NP, 32), src, spad, dstr, z32).reshape(2, R32, 128)

    sp, bsrc, tp, btgt = _head(z3, xw3, dinv32, bg3k, maskp,
                               Ws1k, bs1k, Ws2k, bs2k,
                               Wt1ak, bt1k, Wt1b4, Wt2k, bt2k)
    return (sp.reshape(NP, 1)[:N], bsrc.reshape(()),
            tp.reshape(NP, 1)[:N], btgt.reshape(()))


# speedup vs baseline: 1.0108x; 1.0108x over previous
import functools

import jax
import jax.numpy as jnp
from jax import lax
from jax.experimental import pallas as pl
from jax.experimental.pallas import tpu as pltpu
from jax.experimental.pallas import tpu_sc as plsc

N = 10000
NP = 10112
SLICE = NP // 16
MAXN = 9993
E = 160000
CHUNK = 128
EP = 163840
NROWS = EP // CHUNK
RPT = NROWS // 32
SUP = 8
DUMP = N

R8 = NP * 8 // 128
R16 = NP * 16 // 128
R32 = NP * 32 // 128


@functools.lru_cache(maxsize=None)
def _mesh():
    return plsc.VectorSubcoreMesh(core_axis_name="c", subcore_axis_name="s")


def _relu6(v):
    return jnp.clip(v, 0.0, 6.0)


def _mm(a, b):
    return jax.lax.dot_general(a, b, (((1,), (0,)), ((), ())),
                               preferred_element_type=jnp.float32)


@functools.lru_cache(maxsize=None)
def _make_deg():
    @functools.partial(
        pl.kernel,
        out_type=jax.ShapeDtypeStruct((2, NP, 16), jnp.float32),
        mesh=_mesh(),
        scratch_types=[
            pltpu.VMEM((RPT, CHUNK), jnp.int32),
            pltpu.VMEM((CHUNK, 16), jnp.float32),
            pltpu.VMEM_SHARED((NP, 16), jnp.float32),
            pltpu.SemaphoreType.DMA,
        ],
        compiler_params=pltpu.CompilerParams(use_tc_tiling_on_sc=False),
    )
    def deg_kernel(dstr_hbm, ones_hbm, zeros_hbm, out_hbm, didx, ones_v, acc,
                   sem):
        c = lax.axis_index("c")
        s = lax.axis_index("s")
        tile = s * 2 + c
        pltpu.sync_copy(zeros_hbm.at[pl.ds(s * SLICE, SLICE)],
                        acc.at[pl.ds(s * SLICE, SLICE)])
        pltpu.sync_copy(dstr_hbm.at[pl.ds(tile * RPT, RPT)], didx)
        pltpu.sync_copy(ones_hbm, ones_v)
        plsc.subcore_barrier()
        handles = []
        for g in range(RPT):
            handles.append(
                pltpu.async_copy(ones_v, acc.at[didx.at[g]], sem, add=True))
            if len(handles) > 8:
                handles.pop(0).wait()
        for h in handles:
            h.wait()
        plsc.subcore_barrier()
        pltpu.sync_copy(acc.at[pl.ds(s * SLICE, SLICE)],
                        out_hbm.at[c, pl.ds(s * SLICE, SLICE)])

    return deg_kernel


@functools.lru_cache(maxsize=None)
def _make_segsum(F):
    @functools.partial(
        pl.kernel,
        out_type=jax.ShapeDtypeStruct((2, NP, F), jnp.float32),
        mesh=_mesh(),
        scratch_types=[
            pltpu.VMEM((RPT * CHUNK,), jnp.int32),
            pltpu.VMEM((RPT, CHUNK), jnp.int32),
            pltpu.VMEM((SUP * CHUNK, F), jnp.float32),
            pltpu.VMEM((SUP * CHUNK, F), jnp.float32),
            pltpu.VMEM((SUP * CHUNK, F), jnp.float32),
            pltpu.VMEM_SHARED((NP, F), jnp.float32),
            pltpu.SemaphoreType.DMA,
            pltpu.SemaphoreType.DMA,
            pltpu.SemaphoreType.DMA,
            pltpu.SemaphoreType.DMA,
            pltpu.SemaphoreType.DMA,
            pltpu.SemaphoreType.DMA,
        ],
        compiler_params=pltpu.CompilerParams(use_tc_tiling_on_sc=False),
    )
    def seg_kernel(y_hbm, srcf_hbm, dstr_hbm, zeros_hbm, out_hbm,
                   sidx, didx, buf_a, buf_b, buf_c, acc,
                   gs_a, gs_b, gs_c, ss_a, ss_b, ss_c):
        c = lax.axis_index("c")
        s = lax.axis_index("s")
        tile = s * 2 + c

        pltpu.sync_copy(zeros_hbm.at[pl.ds(s * SLICE, SLICE)],
                        acc.at[pl.ds(s * SLICE, SLICE)])
        pltpu.sync_copy(srcf_hbm.at[pl.ds(tile * RPT * CHUNK, RPT * CHUNK)],
                        sidx)
        pltpu.sync_copy(dstr_hbm.at[pl.ds(tile * RPT, RPT)], didx)
        plsc.subcore_barrier()

        banks = ((buf_a, gs_a, ss_a), (buf_b, gs_b, ss_b),
                 (buf_c, gs_c, ss_c))
        NB = RPT // SUP

        def fire_gather(bank, base):
            buf, gs, _ = banks[bank]
            return [pltpu.async_copy(
                y_hbm.at[sidx.at[pl.ds(base * CHUNK, SUP * CHUNK)]],
                buf, gs)]

        scat_pend = {0: [], 1: [], 2: []}
        hg = {0: fire_gather(0, 0), 1: fire_gather(1, SUP), 2: None}
        for sc in range(NB):
            bank = sc % 3
            buf, _, ss = banks[bank]
            base = sc * SUP
            for h in hg[bank]:
                h.wait()
            if sc + 2 < NB:
                nbank = (sc + 2) % 3
                for h in scat_pend[nbank]:
                    h.wait()
                scat_pend[nbank] = []
                hg[nbank] = fire_gather(nbank, base + 2 * SUP)
            scat_pend[bank] = [
                pltpu.async_copy(buf.at[pl.ds(j * CHUNK, CHUNK)],
                                 acc.at[didx.at[base + j]],
                                 ss, add=True) for j in range(SUP)]
        for b in (0, 1, 2):
            for h in scat_pend[b]:
                h.wait()
        plsc.subcore_barrier()
        pltpu.sync_copy(acc.at[pl.ds(s * SLICE, SLICE)],
                        out_hbm.at[c, pl.ds(s * SLICE, SLICE)])

    return seg_kernel


def _deg_sc(*a):
    return _make_deg()(*a)


def _seg16(*a):
    return _make_segsum(16)(*a)


def _seg32(*a):
    return _make_segsum(32)(*a)


def _tc1_body(feat_ref, degs_ref, w0k_ref, b0k_ref, w1k_ref, t32_ref,
              xw1_ref, y1_ref, dinv16_ref, dinv32_ref):
    deg = degs_ref[0] + degs_ref[1] + 1.0
    dinv16 = lax.rsqrt(deg)
    x0 = _relu6(_mm(feat_ref[...], w0k_ref[...]) + b0k_ref[...])
    xw1 = jnp.reshape(_mm(x0, w1k_ref[...]), (R16, 128))
    deg32 = jnp.reshape(_mm(deg, t32_ref[...]), (R32, 128))
    dinv32 = lax.rsqrt(deg32)
    xw1_ref[...] = xw1
    y1_ref[...] = xw1 * dinv16
    dinv16_ref[...] = dinv16
    dinv32_ref[...] = dinv32


def _tc1(featp, degsp, W0k, b0k, Wg1k, T32k):
    return pl.pallas_call(
        _tc1_body,
        out_shape=[
            jax.ShapeDtypeStruct((R16, 128), jnp.float32),
            jax.ShapeDtypeStruct((R16, 128), jnp.float32),
            jax.ShapeDtypeStruct((R16, 128), jnp.float32),
            jax.ShapeDtypeStruct((R32, 128), jnp.float32),
        ],
    )(featp, degsp, W0k, b0k, Wg1k, T32k)


def _tc_layer_body(z_ref, xw_ref, dinv_ref, dinvn_ref, bk_ref, wnk_ref,
                   xwn_ref, yn_ref):
    dinv = dinv_ref[...]
    x = _relu6((z_ref[0] + z_ref[1] + xw_ref[...] * dinv) * dinv
               + bk_ref[...])
    xwn = jnp.reshape(_mm(x, wnk_ref[...]), (R32, 128))
    xwn_ref[...] = xwn
    yn_ref[...] = xwn * dinvn_ref[...]


def _tc_layer(zp, xwp, dinvp, dinvnp, bk, Wnk):
    return pl.pallas_call(
        _tc_layer_body,
        out_shape=[
            jax.ShapeDtypeStruct((R32, 128), jnp.float32),
            jax.ShapeDtypeStruct((R32, 128), jnp.float32),
        ],
    )(zp, xwp, dinvp, dinvnp, bk, Wnk)


def _head_body(z_ref, xw_ref, dinv_ref, bgk_ref, mask_ref,
               ws1k_ref, bs1k_ref, ws2k_ref, bs2k_ref,
               wt1ak_ref, bt1k_ref, wt1b4_ref, wt2k_ref, bt2k_ref,
               sp_ref, bsrc_ref, tp_ref, btgt_ref):
    dinv = dinv_ref[...]
    x = _relu6((z_ref[0] + z_ref[1] + xw_ref[...] * dinv) * dinv
               + bgk_ref[...])
    node = (lax.broadcasted_iota(jnp.int32, (R32, 4), 0) * 4
            + lax.broadcasted_iota(jnp.int32, (R32, 4), 1))
    valid = node < N

    hh = _relu6(_mm(x, ws1k_ref[...]) + bs1k_ref[...])
    sl = _mm(hh, ws2k_ref[...]) + bs2k_ref[...]
    slv = jnp.where(valid, sl, -1e30)
    e = jnp.where(valid, jnp.exp(sl - jnp.max(slv)), 0.0)
    sp = e / jnp.sum(e)
    m = mask_ref[...] > 0.0
    sp_ref[...] = jnp.where(m, 0.0, sp)
    am = jnp.where(valid, jnp.where(m, -1.0, sp), -2.0)
    best = jnp.min(jnp.where(am == jnp.max(am), node, NP))
    bsrc_ref[...] = jnp.full((1, 1), best, jnp.int32)

    rowsel = lax.broadcasted_iota(jnp.int32, (R32, 128), 0) == (best // 4)
    xsrow = jnp.sum(jnp.where(rowsel, x, 0.0), axis=0, keepdims=True)
    gsel = (lax.broadcasted_iota(jnp.int32, (1, 128), 1) // 32) == (best % 4)
    xsw = _mm(jnp.where(gsel, xsrow, 0.0), wt1b4_ref[...])
    xsw4 = jnp.concatenate([xsw, xsw, xsw, xsw], axis=1)
    th = _relu6(_mm(x, wt1ak_ref[...]) + xsw4 + bt1k_ref[...])
    tl = _mm(th, wt2k_ref[...]) + bt2k_ref[...]
    tlv = jnp.where(valid, tl, -1e30)
    te = jnp.where(valid, jnp.exp(tl - jnp.max(tlv)), 0.0)
    tp = te / jnp.sum(te)
    tvalid = node < MAXN
    tp_ref[...] = jnp.where(tvalid, tp, 0.0)
    am2 = jnp.where(tvalid, tp, -1.0)
    best2 = jnp.min(jnp.where(am2 == jnp.max(am2), node, NP))
    btgt_ref[...] = jnp.full((1, 1), best2, jnp.int32)


def _head(zp, xwp, dinvp, bgk, maskp, Ws1k, bs1k, Ws2k, bs2k,
          Wt1ak, bt1k, Wt1b4, Wt2k, bt2k):
    return pl.pallas_call(
        _head_body,
        out_shape=[
            jax.ShapeDtypeStruct((R32, 4), jnp.float32),
            jax.ShapeDtypeStruct((1, 1), jnp.int32),
            jax.ShapeDtypeStruct((R32, 4), jnp.float32),
            jax.ShapeDtypeStruct((1, 1), jnp.int32),
        ],
    )(zp, xwp, dinvp, bgk, maskp, Ws1k, bs1k, Ws2k, bs2k,
      Wt1ak, bt1k, Wt1b4, Wt2k, bt2k)


def _kron(k, W):
    return jnp.kron(jnp.eye(k, dtype=jnp.float32), W)


def kernel(feat, edge_index, mask_candidate_set,
           W0, b0, Wg1, bg1, Wg2, bg2, Wg3, bg3,
           Ws1, bs1, Ws2, bs2, Wt1, bt1, Wt2, bt2):
    src = edge_index[0].astype(jnp.int32)
    dst = edge_index[1].astype(jnp.int32)
    dump = jnp.broadcast_to(
        DUMP + (jnp.arange(EP // 32 - E // 32, dtype=jnp.int32) % (NP - N)),
        (32, EP // 32 - E // 32))
    srcf = jnp.concatenate([src.reshape(32, E // 32), dump],
                           axis=1).reshape(EP)
    dstr = jnp.concatenate([dst.reshape(32, E // 32), dump],
                           axis=1).reshape(NROWS, CHUNK)

    featp = jnp.pad(feat, ((0, NP - N), (0, 1))).reshape(R8, 128)
    maskp = jnp.pad(mask_candidate_set.astype(jnp.float32),
                    (0, NP - N)).reshape(R32, 4)
    ones_c = jnp.ones((CHUNK, 16), jnp.float32)
    z16 = jnp.zeros((NP, 16), jnp.float32)
    z32 = jnp.zeros((NP, 32), jnp.float32)

    W0p = jnp.pad(W0, ((0, 1), (0, 0)))
    W0k = _kron(16, W0p)
    b0k = jnp.tile(b0, 16)
    Wg1k = _kron(16, Wg1)
    bg1k = jnp.tile(bg1, 8)
    T = jnp.zeros((16, 32), jnp.float32).at[0, :].set(1.0)
    T32k = _kron(8, T)
    Wg2p = jnp.pad(Wg2, ((0, 0), (0, 8)))
    Wg2k = _kron(8, Wg2p)
    bg2k = jnp.tile(jnp.pad(bg2, (0, 8)), 4)
    Wg3p = jnp.pad(Wg3, ((0, 8), (0, 0)))
    Wg3k = _kron(4, Wg3p)
    bg3k = jnp.tile(bg3, 4)
    Ws1k = _kron(4, Ws1)
    bs1k = jnp.tile(bs1, 4)
    Ws2k = _kron(4, Ws2)
    bs2k = jnp.tile(bs2, 4)
    Wt1ak = _kron(4, Wt1[:32])
    bt1k = jnp.tile(bt1, 4)
    Wt1b4 = jnp.concatenate([Wt1[32:]] * 4, axis=0)
    Wt2k = _kron(4, Wt2)
    bt2k = jnp.tile(bt2, 4)

    degs = _deg_sc(dstr, ones_c, z16)
    degsp = degs.reshape(2, R16, 128)
    xw1, y1, dinv16, dinv32 = _tc1(featp, degsp, W0k, b0k, Wg1k, T32k)
    z1 = _seg16(y1.reshape(NP, 16), srcf, dstr, z16).reshape(2, R16, 128)
    xw2, y2 = _tc_layer(z1, xw1, dinv16, dinv32, bg1k, Wg2k)
    z2 = _seg32(y2.reshape(NP, 32), srcf, dstr, z32).reshape(2, R32, 128)
    xw3, y3 = _tc_layer(z2, xw2, dinv32, dinv32, bg2k, Wg3k)
    z3 = _seg32(y3.reshape(NP, 32), srcf, dstr, z32).reshape(2, R32, 128)

    sp, bsrc, tp, btgt = _head(z3, xw3, dinv32, bg3k, maskp,
                               Ws1k, bs1k, Ws2k, bs2k,
                               Wt1ak, bt1k, Wt1b4, Wt2k, bt2k)
    return (sp.reshape(NP, 1)[:N], bsrc.reshape(()),
            tp.reshape(NP, 1)[:N], btgt.reshape(()))
